# Initial kernel scaffold; baseline (speedup 1.0000x reference)
import jax
import jax.numpy as jnp
from jax.experimental import pallas as pl


def kernel(x, gamma, beta, emb, modality_idx):
    raise NotImplementedError("write your pallas kernel here")


# speedup vs baseline: 1.8656x; 1.8656x over previous
import jax
import jax.numpy as jnp
from jax.experimental import pallas as pl
from jax.experimental.pallas import tpu as pltpu

DIM = 2048
EPS = 1e-5
BLOCK_ROWS = 512


def _ln_add_kernel(idx_ref, x_ref, gamma_ref, beta_ref, emb_ref, o_ref):
    x = x_ref[...]
    mean = jnp.mean(x, axis=-1, keepdims=True)
    xc = x - mean
    var = jnp.mean(xc * xc, axis=-1, keepdims=True)
    inv = jax.lax.rsqrt(var + EPS)
    e = emb_ref[idx_ref[0], :]
    o_ref[...] = xc * inv * gamma_ref[...] + (beta_ref[...] + e)


def kernel(x, gamma, beta, emb, modality_idx):
    orig_shape = x.shape
    rows = x.size // DIM
    x2 = x.reshape(rows, DIM)
    grid = (rows // BLOCK_ROWS,)
    idx = jnp.reshape(modality_idx, (1,)).astype(jnp.int32)

    out = pl.pallas_call(
        _ln_add_kernel,
        grid_spec=pltpu.PrefetchScalarGridSpec(
            num_scalar_prefetch=1,
            grid=grid,
            in_specs=[
                pl.BlockSpec((BLOCK_ROWS, DIM), lambda i, s: (i, 0)),
                pl.BlockSpec((DIM,), lambda i, s: (0,)),
                pl.BlockSpec((DIM,), lambda i, s: (0,)),
                pl.BlockSpec(emb.shape, lambda i, s: (0, 0)),
            ],
            out_specs=pl.BlockSpec((BLOCK_ROWS, DIM), lambda i, s: (i, 0)),
        ),
        out_shape=jax.ShapeDtypeStruct((rows, DIM), x.dtype),
    )(idx, x2, gamma, beta, emb)
    return out.reshape(orig_shape)


# speedup vs baseline: 1.9144x; 1.0261x over previous
import jax
import jax.numpy as jnp
from jax.experimental import pallas as pl
from jax.experimental.pallas import tpu as pltpu

DIM = 2048
EPS = 1e-5
BLOCK_ROWS = 1024


def _ln_add_kernel(idx_ref, x_ref, gamma_ref, beta_ref, emb_ref, o_ref):
    x = x_ref[...]
    mean = jnp.mean(x, axis=-1, keepdims=True)
    xc = x - mean
    var = jnp.mean(xc * xc, axis=-1, keepdims=True)
    inv = jax.lax.rsqrt(var + EPS)
    e = emb_ref[idx_ref[0], :]
    o_ref[...] = xc * inv * gamma_ref[...] + (beta_ref[...] + e)


def kernel(x, gamma, beta, emb, modality_idx):
    orig_shape = x.shape
    rows = x.size // DIM
    x2 = x.reshape(rows, DIM)
    grid = (rows // BLOCK_ROWS,)
    idx = jnp.reshape(modality_idx, (1,)).astype(jnp.int32)

    out = pl.pallas_call(
        _ln_add_kernel,
        grid_spec=pltpu.PrefetchScalarGridSpec(
            num_scalar_prefetch=1,
            grid=grid,
            in_specs=[
                pl.BlockSpec((BLOCK_ROWS, DIM), lambda i, s: (i, 0)),
                pl.BlockSpec((DIM,), lambda i, s: (0,)),
                pl.BlockSpec((DIM,), lambda i, s: (0,)),
                pl.BlockSpec(emb.shape, lambda i, s: (0, 0)),
            ],
            out_specs=pl.BlockSpec((BLOCK_ROWS, DIM), lambda i, s: (i, 0)),
        ),
        out_shape=jax.ShapeDtypeStruct((rows, DIM), x.dtype),
    )(idx, x2, gamma, beta, emb)
    return out.reshape(orig_shape)
